# Initial kernel scaffold; baseline (speedup 1.0000x reference)
import jax
import jax.numpy as jnp
from jax.experimental import pallas as pl


def kernel(weights, gst):
    raise NotImplementedError("write your pallas kernel here")


# speedup vs baseline: 64.1524x; 64.1524x over previous
import jax
import jax.numpy as jnp
from jax.experimental import pallas as pl
from jax.experimental.pallas import tpu as pltpu

_TB = 2048


def _style_matmul_kernel(w_ref, gst_ref, out_ref):
    out_ref[...] = jnp.dot(
        w_ref[...], gst_ref[...], preferred_element_type=jnp.float32
    ).astype(out_ref.dtype)


def kernel(weights, gst):
    B, T = weights.shape
    T2, D = gst.shape
    assert T == T2
    tb = min(_TB, B)
    grid = (pl.cdiv(B, tb),)
    return pl.pallas_call(
        _style_matmul_kernel,
        out_shape=jax.ShapeDtypeStruct((B, D), jnp.float32),
        grid_spec=pltpu.PrefetchScalarGridSpec(
            num_scalar_prefetch=0,
            grid=grid,
            in_specs=[
                pl.BlockSpec((tb, T), lambda i: (i, 0)),
                pl.BlockSpec((T2, D), lambda i: (0, 0)),
            ],
            out_specs=pl.BlockSpec((tb, D), lambda i: (i, 0)),
        ),
        compiler_params=pltpu.CompilerParams(dimension_semantics=("parallel",)),
    )(weights, gst)


# speedup vs baseline: 71.1908x; 1.1097x over previous
import jax
import jax.numpy as jnp
from jax.experimental import pallas as pl
from jax.experimental.pallas import tpu as pltpu

_TB = 4096


def _style_matmul_kernel(w_ref, gst_ref, out_ref):
    out_ref[...] = jnp.dot(
        w_ref[...], gst_ref[...], preferred_element_type=jnp.float32
    ).astype(out_ref.dtype)


def kernel(weights, gst):
    B, T = weights.shape
    T2, D = gst.shape
    assert T == T2
    tb = min(_TB, B)
    grid = (pl.cdiv(B, tb),)
    return pl.pallas_call(
        _style_matmul_kernel,
        out_shape=jax.ShapeDtypeStruct((B, D), jnp.float32),
        grid_spec=pltpu.PrefetchScalarGridSpec(
            num_scalar_prefetch=0,
            grid=grid,
            in_specs=[
                pl.BlockSpec((tb, T), lambda i: (i, 0)),
                pl.BlockSpec((T2, D), lambda i: (0, 0)),
            ],
            out_specs=pl.BlockSpec((tb, D), lambda i: (i, 0)),
        ),
        compiler_params=pltpu.CompilerParams(dimension_semantics=("parallel",)),
    )(weights, gst)


# speedup vs baseline: 72.6255x; 1.0202x over previous
import jax
import jax.numpy as jnp
from jax.experimental import pallas as pl
from jax.experimental.pallas import tpu as pltpu

_TB = 8192


def _style_matmul_kernel(w_ref, gst_ref, out_ref):
    out_ref[...] = jnp.dot(
        w_ref[...], gst_ref[...], preferred_element_type=jnp.float32
    ).astype(out_ref.dtype)


def kernel(weights, gst):
    B, T = weights.shape
    T2, D = gst.shape
    assert T == T2
    tb = min(_TB, B)
    grid = (pl.cdiv(B, tb),)
    return pl.pallas_call(
        _style_matmul_kernel,
        out_shape=jax.ShapeDtypeStruct((B, D), jnp.float32),
        grid_spec=pltpu.PrefetchScalarGridSpec(
            num_scalar_prefetch=0,
            grid=grid,
            in_specs=[
                pl.BlockSpec((tb, T), lambda i: (i, 0)),
                pl.BlockSpec((T2, D), lambda i: (0, 0)),
            ],
            out_specs=pl.BlockSpec((tb, D), lambda i: (i, 0)),
        ),
        compiler_params=pltpu.CompilerParams(dimension_semantics=("parallel",)),
    )(weights, gst)
